# Initial kernel scaffold; baseline (speedup 1.0000x reference)
import jax
import jax.numpy as jnp
from jax.experimental import pallas as pl


def kernel(Title, FullDescription, Categorical, embedding_matrix, W_cat, b_cat, W_out, b_out):
    raise NotImplementedError("write your pallas kernel here")


# speedup vs baseline: 74.3560x; 74.3560x over previous
import functools

import jax
import jax.numpy as jnp
from jax import lax
from jax.experimental import pallas as pl
from jax.experimental.pallas import tpu as pltpu
from jax.experimental.pallas import tpu_sc as plsc

B = 16384
LT = 20
LD = 200
V = 100000
D = 64
C = 100
H = 128

NC = 2
NS = 16
NW = NC * NS
RPW = B // NW
SUB = 8
RSUB = RPW // SUB


def _scores_body(e_ref, w2_ref, out_ref):
    out_ref[...] = lax.dot_general(
        w2_ref[...], e_ref[...],
        dimension_numbers=(((1,), (1,)), ((), ())),
        preferred_element_type=jnp.float32)


def _compute_scores(E, w2):
    VB = 8192
    return pl.pallas_call(
        _scores_body,
        grid=(pl.cdiv(V, VB),),
        in_specs=[
            pl.BlockSpec((VB, D), lambda i: (i, 0)),
            pl.BlockSpec((2, D), lambda i: (0, 0)),
        ],
        out_specs=pl.BlockSpec((2, VB), lambda i: (0, i)),
        out_shape=jax.ShapeDtypeStruct((2, V), jnp.float32),
    )(E, w2)


def _sc_pool(s2, title_flat, desc_flat):
    mesh = plsc.VectorSubcoreMesh(core_axis_name="c", subcore_axis_name="s")

    @functools.partial(
        pl.kernel,
        mesh=mesh,
        out_type=jax.ShapeDtypeStruct((B,), jnp.float32),
        compiler_params=pltpu.CompilerParams(needs_layout_passes=False),
        scratch_types=[
            pltpu.VMEM((V,), jnp.float32),
            pltpu.VMEM((RSUB * LD,), jnp.int32),
            pltpu.VMEM((RPW,), jnp.float32),
        ],
    )
    def k(s2_hbm, title_hbm, desc_hbm, out_hbm, table_v, idx_v, acc_v):
        cid = lax.axis_index("c")
        sid = lax.axis_index("s")
        w = cid * NS + sid
        r0 = w * RPW
        lanes = lax.iota(jnp.int32, 16)

        pltpu.sync_copy(s2_hbm.at[0], table_v)
        pltpu.sync_copy(title_hbm.at[pl.ds(r0 * LT, RPW * LT)],
                        idx_v.at[pl.ds(0, RPW * LT)])

        def title_group(g, carry):
            base = (g * 16 + lanes) * LT
            acc0 = jnp.zeros((16,), jnp.float32)
            acc1 = jnp.zeros((16,), jnp.float32)
            for t in range(LT):
                iv = plsc.load_gather(idx_v, [base + t])
                vals = plsc.load_gather(table_v, [iv])
                if t % 2 == 0:
                    acc0 = acc0 + vals
                else:
                    acc1 = acc1 + vals
            acc_v[pl.ds(g * 16, 16)] = acc0 + acc1
            return carry

        lax.fori_loop(0, RPW // 16, title_group, 0)

        pltpu.sync_copy(s2_hbm.at[1], table_v)

        def sub_loop(sub, carry):
            pltpu.sync_copy(
                desc_hbm.at[pl.ds((r0 + sub * RSUB) * LD, RSUB * LD)], idx_v)

            def desc_group(g, c2):
                base = (g * 16 + lanes) * LD
                accs = [jnp.zeros((16,), jnp.float32) for _ in range(4)]
                for t in range(LD):
                    iv = plsc.load_gather(idx_v, [base + t])
                    vals = plsc.load_gather(table_v, [iv])
                    accs[t % 4] = accs[t % 4] + vals
                tot = (accs[0] + accs[1]) + (accs[2] + accs[3])
                off = sub * RSUB + g * 16
                acc_v[pl.ds(off, 16)] = acc_v[pl.ds(off, 16)] + tot
                return c2

            lax.fori_loop(0, RSUB // 16, desc_group, 0)
            return carry

        lax.fori_loop(0, SUB, sub_loop, 0)

        pltpu.sync_copy(acc_v, out_hbm.at[pl.ds(r0, RPW)])

    return k(s2, title_flat, desc_flat)


def _final_body(cat_ref, wcat_ref, bcat_ref, wh_ref, bout_ref, emb_ref,
                out_ref):
    h = lax.dot_general(
        cat_ref[...], wcat_ref[...],
        dimension_numbers=(((1,), (1,)), ((), ())),
        preferred_element_type=jnp.float32)
    h = jnp.maximum(h + bcat_ref[...], 0.0)
    o = lax.dot_general(
        wh_ref[...], h,
        dimension_numbers=(((1,), (1,)), ((), ())),
        preferred_element_type=jnp.float32)
    out_ref[...] = o + emb_ref[...] + bout_ref[0, 0]


def _final(Cat, W_cat, b_cat2, wh2, b_out2, emb2):
    BB = 2048
    return pl.pallas_call(
        _final_body,
        grid=(B // BB,),
        in_specs=[
            pl.BlockSpec((BB, C), lambda i: (i, 0)),
            pl.BlockSpec((H, C), lambda i: (0, 0)),
            pl.BlockSpec((1, H), lambda i: (0, 0)),
            pl.BlockSpec((1, H), lambda i: (0, 0)),
            pl.BlockSpec((1, 1), lambda i: (0, 0)),
            pl.BlockSpec((1, BB), lambda i: (0, i)),
        ],
        out_specs=pl.BlockSpec((1, BB), lambda i: (0, i)),
        out_shape=jax.ShapeDtypeStruct((1, B), jnp.float32),
    )(Cat, W_cat, b_cat2, wh2, b_out2, emb2)


def kernel(Title, FullDescription, Categorical, embedding_matrix, W_cat,
           b_cat, W_out, b_out):
    w2 = jnp.stack([W_out[0, :D] * (1.0 / LT),
                    W_out[0, D:2 * D] * (1.0 / LD)], axis=0)
    s2 = _compute_scores(embedding_matrix, w2)

    title_flat = Title.reshape(-1).astype(jnp.int32)
    desc_flat = FullDescription.reshape(-1).astype(jnp.int32)
    emb_part = _sc_pool(s2, title_flat, desc_flat)

    out2 = _final(Categorical, W_cat, b_cat.reshape(1, H),
                  W_out[0:1, 2 * D:], b_out.reshape(1, 1),
                  emb_part.reshape(1, B))
    return out2.reshape(B)


# speedup vs baseline: 78.7205x; 1.0587x over previous
import functools

import jax
import jax.numpy as jnp
from jax import lax
from jax.experimental import pallas as pl
from jax.experimental.pallas import tpu as pltpu
from jax.experimental.pallas import tpu_sc as plsc

B = 16384
LT = 20
LD = 200
V = 100000
D = 64
C = 100
H = 128

NC = 2
NS = 16
NW = NC * NS
RPW = B // NW
SUB = 16
RSUB = RPW // SUB
TA = RSUB * LD // LT


def _scores_body(e_ref, w2_ref, out_ref):
    out_ref[...] = lax.dot_general(
        w2_ref[...], e_ref[...],
        dimension_numbers=(((1,), (1,)), ((), ())),
        preferred_element_type=jnp.float32)


def _compute_scores(E, w2):
    VB = 8192
    return pl.pallas_call(
        _scores_body,
        grid=(pl.cdiv(V, VB),),
        in_specs=[
            pl.BlockSpec((VB, D), lambda i: (i, 0)),
            pl.BlockSpec((2, D), lambda i: (0, 0)),
        ],
        out_specs=pl.BlockSpec((2, VB), lambda i: (0, i)),
        out_shape=jax.ShapeDtypeStruct((2, V), jnp.float32),
    )(E, w2)


def _sc_pool(s2, title_flat, desc_flat):
    mesh = plsc.VectorSubcoreMesh(core_axis_name="c", subcore_axis_name="s")

    @functools.partial(
        pl.kernel,
        mesh=mesh,
        out_type=jax.ShapeDtypeStruct((B,), jnp.float32),
        compiler_params=pltpu.CompilerParams(needs_layout_passes=False),
        scratch_types=[
            pltpu.VMEM((V,), jnp.float32),
            pltpu.VMEM((RSUB * LD,), jnp.int32),
            pltpu.VMEM((RSUB * LD,), jnp.int32),
            pltpu.VMEM((RPW,), jnp.float32),
            pltpu.SemaphoreType.DMA,
            pltpu.SemaphoreType.DMA,
            pltpu.SemaphoreType.DMA,
        ],
    )
    def k(s2_hbm, title_hbm, desc_hbm, out_hbm, table_v, buf_a, buf_b,
          acc_v, semT, semA, semB):
        w = lax.axis_index("c") * NS + lax.axis_index("s")
        r0 = w * RPW
        lanes = lax.iota(jnp.int32, 16)

        tcp = pltpu.make_async_copy(s2_hbm.at[0], table_v, semT)
        tcp.start()
        ta = pltpu.make_async_copy(
            title_hbm.at[pl.ds(r0 * LT, TA * LT)], buf_a, semA)
        ta.start()
        tb = pltpu.make_async_copy(
            title_hbm.at[pl.ds((r0 + TA) * LT, (RPW - TA) * LT)],
            buf_b.at[pl.ds(0, (RPW - TA) * LT)], semB)
        tb.start()
        tcp.wait()
        ta.wait()
        tb.wait()

        def make_title_group(buf, g0):
            def title_group(g, carry):
                base = ((g - g0) * 16 + lanes) * LT
                acc0 = jnp.zeros((16,), jnp.float32)
                acc1 = jnp.zeros((16,), jnp.float32)
                for t in range(LT):
                    iv = plsc.load_gather(buf, [base + t])
                    vals = plsc.load_gather(table_v, [iv])
                    if t % 2 == 0:
                        acc0 = acc0 + vals
                    else:
                        acc1 = acc1 + vals
                acc_v[pl.ds(g * 16, 16)] = acc0 + acc1
                return carry
            return title_group

        lax.fori_loop(0, TA // 16, make_title_group(buf_a, 0), 0)
        lax.fori_loop(TA // 16, RPW // 16, make_title_group(buf_b, TA // 16), 0)

        tB = pltpu.make_async_copy(s2_hbm.at[1], table_v, semT)
        tB.start()
        pltpu.make_async_copy(
            desc_hbm.at[pl.ds(r0 * LD, RSUB * LD)], buf_a, semA).start()
        pltpu.make_async_copy(
            desc_hbm.at[pl.ds((r0 + RSUB) * LD, RSUB * LD)], buf_b,
            semB).start()
        tB.wait()

        def desc_compute(buf, sub):
            def group(g, c2):
                base = (g * 16 + lanes) * LD
                accs = [jnp.zeros((16,), jnp.float32) for _ in range(4)]
                for t in range(LD):
                    iv = plsc.load_gather(buf, [base + t])
                    vals = plsc.load_gather(table_v, [iv])
                    accs[t % 4] = accs[t % 4] + vals
                tot = (accs[0] + accs[1]) + (accs[2] + accs[3])
                off = sub * RSUB + g * 16
                acc_v[pl.ds(off, 16)] = acc_v[pl.ds(off, 16)] + tot
                return c2

            lax.fori_loop(0, RSUB // 16, group, 0)

        def pair(p, carry):
            sub = 2 * p
            pltpu.make_async_copy(
                desc_hbm.at[pl.ds(r0 * LD, RSUB * LD)], buf_a, semA).wait()
            desc_compute(buf_a, sub)

            @pl.when(p < SUB // 2 - 1)
            def _():
                pltpu.make_async_copy(
                    desc_hbm.at[pl.ds((r0 + (sub + 2) * RSUB) * LD,
                                      RSUB * LD)], buf_a, semA).start()

            pltpu.make_async_copy(
                desc_hbm.at[pl.ds(r0 * LD, RSUB * LD)], buf_b, semB).wait()
            desc_compute(buf_b, sub + 1)

            @pl.when(p < SUB // 2 - 1)
            def _():
                pltpu.make_async_copy(
                    desc_hbm.at[pl.ds((r0 + (sub + 3) * RSUB) * LD,
                                      RSUB * LD)], buf_b, semB).start()

            return carry

        lax.fori_loop(0, SUB // 2, pair, 0)

        pltpu.sync_copy(acc_v, out_hbm.at[pl.ds(r0, RPW)])

    return k(s2, title_flat, desc_flat)


def _final_body(cat_ref, wcat_ref, bcat_ref, wh_ref, bout_ref, emb_ref,
                out_ref):
    h = lax.dot_general(
        cat_ref[...], wcat_ref[...],
        dimension_numbers=(((1,), (1,)), ((), ())),
        preferred_element_type=jnp.float32)
    h = jnp.maximum(h + bcat_ref[...], 0.0)
    o = lax.dot_general(
        wh_ref[...], h,
        dimension_numbers=(((1,), (1,)), ((), ())),
        preferred_element_type=jnp.float32)
    out_ref[...] = o + emb_ref[...] + bout_ref[0, 0]


def _final(Cat, W_cat, b_cat2, wh2, b_out2, emb2):
    BB = 2048
    return pl.pallas_call(
        _final_body,
        grid=(B // BB,),
        in_specs=[
            pl.BlockSpec((BB, C), lambda i: (i, 0)),
            pl.BlockSpec((H, C), lambda i: (0, 0)),
            pl.BlockSpec((1, H), lambda i: (0, 0)),
            pl.BlockSpec((1, H), lambda i: (0, 0)),
            pl.BlockSpec((1, 1), lambda i: (0, 0)),
            pl.BlockSpec((1, BB), lambda i: (0, i)),
        ],
        out_specs=pl.BlockSpec((1, BB), lambda i: (0, i)),
        out_shape=jax.ShapeDtypeStruct((1, B), jnp.float32),
    )(Cat, W_cat, b_cat2, wh2, b_out2, emb2)


def kernel(Title, FullDescription, Categorical, embedding_matrix, W_cat,
           b_cat, W_out, b_out):
    w2 = jnp.stack([W_out[0, :D] * (1.0 / LT),
                    W_out[0, D:2 * D] * (1.0 / LD)], axis=0)
    s2 = _compute_scores(embedding_matrix, w2)

    title_flat = Title.reshape(-1).astype(jnp.int32)
    desc_flat = FullDescription.reshape(-1).astype(jnp.int32)
    emb_part = _sc_pool(s2, title_flat, desc_flat)

    out2 = _final(Categorical, W_cat, b_cat.reshape(1, H),
                  W_out[0:1, 2 * D:], b_out.reshape(1, 1),
                  emb_part.reshape(1, B))
    return out2.reshape(B)


# speedup vs baseline: 80.4611x; 1.0221x over previous
import functools

import jax
import jax.numpy as jnp
from jax import lax
from jax.experimental import pallas as pl
from jax.experimental.pallas import tpu as pltpu
from jax.experimental.pallas import tpu_sc as plsc

B = 16384
LT = 20
LD = 200
V = 100000
D = 64
C = 100
H = 128

NC = 2
NS = 16
NW = NC * NS
RPW = B // NW
SUB = 32
RSUB = RPW // SUB


def _scores_body(e_ref, w2_ref, out_ref):
    out_ref[...] = lax.dot_general(
        w2_ref[...], e_ref[...],
        dimension_numbers=(((1,), (1,)), ((), ())),
        preferred_element_type=jnp.float32)


def _compute_scores(E, w2):
    VB = 8192
    return pl.pallas_call(
        _scores_body,
        grid=(pl.cdiv(V, VB),),
        in_specs=[
            pl.BlockSpec((VB, D), lambda i: (i, 0)),
            pl.BlockSpec((2, D), lambda i: (0, 0)),
        ],
        out_specs=pl.BlockSpec((2, VB), lambda i: (0, i)),
        out_shape=jax.ShapeDtypeStruct((2, V), jnp.float32),
    )(E, w2)


def _sc_pool(s2, title_flat, desc2d):
    mesh = plsc.VectorSubcoreMesh(core_axis_name="c", subcore_axis_name="s")

    @functools.partial(
        pl.kernel,
        mesh=mesh,
        out_type=jax.ShapeDtypeStruct((B,), jnp.float32),
        compiler_params=pltpu.CompilerParams(needs_layout_passes=False),
        scratch_types=[
            pltpu.VMEM((V,), jnp.float32),
            pltpu.VMEM((RPW * LT,), jnp.int32),
            pltpu.VMEM((RSUB, LD), jnp.int32),
            pltpu.VMEM((RSUB, LD), jnp.int32),
            pltpu.VMEM((RPW,), jnp.float32),
            pltpu.SemaphoreType.DMA,
            pltpu.SemaphoreType.DMA,
            pltpu.SemaphoreType.DMA,
        ],
    )
    def k(s2_hbm, title_hbm, desc_hbm, out_hbm, table_v, tidx_v, buf_a,
          buf_b, acc_v, semT, semA, semB):
        w = lax.axis_index("c") * NS + lax.axis_index("s")
        r0 = w * RPW
        lanes = lax.iota(jnp.int32, 16)

        tcp = pltpu.make_async_copy(s2_hbm.at[0], table_v, semT)
        tcp.start()
        ti = pltpu.make_async_copy(
            title_hbm.at[pl.ds(r0 * LT, RPW * LT)], tidx_v, semB)
        ti.start()
        pltpu.make_async_copy(
            desc_hbm.at[pl.ds(r0, RSUB), :], buf_a, semA).start()
        tcp.wait()
        ti.wait()

        def title_group(g, carry):
            base = (g * 16 + lanes) * LT
            acc0 = jnp.zeros((16,), jnp.float32)
            acc1 = jnp.zeros((16,), jnp.float32)
            for t in range(LT):
                iv = plsc.load_gather(tidx_v, [base + t])
                vals = plsc.load_gather(table_v, [iv])
                if t % 2 == 0:
                    acc0 = acc0 + vals
                else:
                    acc1 = acc1 + vals
            acc_v[pl.ds(g * 16, 16)] = acc0 + acc1
            return carry

        lax.fori_loop(0, RPW // 16, title_group, 0)

        tB = pltpu.make_async_copy(s2_hbm.at[1], table_v, semT)
        tB.start()
        pltpu.make_async_copy(
            desc_hbm.at[pl.ds(r0 + RSUB, RSUB), :], buf_b, semB).start()
        tB.wait()

        def desc_compute(buf, sub):
            def tchunk(tc, accs):
                t0 = tc * 25
                for j in range(25):
                    iv = plsc.load_gather(
                        buf, [lanes, jnp.full((16,), t0 + j, jnp.int32)])
                    vals = plsc.load_gather(table_v, [iv])
                    accs = tuple(a + vals if i == j % 4 else a
                                 for i, a in enumerate(accs))
                return accs

            z = jnp.zeros((16,), jnp.float32)
            accs = lax.fori_loop(0, LD // 25, tchunk, (z, z, z, z))
            tot = (accs[0] + accs[1]) + (accs[2] + accs[3])
            off = sub * RSUB
            acc_v[pl.ds(off, 16)] = acc_v[pl.ds(off, 16)] + tot

        def pair(p, carry):
            sub = 2 * p
            pltpu.make_async_copy(
                desc_hbm.at[pl.ds(r0, RSUB), :], buf_a, semA).wait()
            desc_compute(buf_a, sub)

            @pl.when(p < SUB // 2 - 1)
            def _():
                pltpu.make_async_copy(
                    desc_hbm.at[pl.ds(r0 + (sub + 2) * RSUB, RSUB), :],
                    buf_a, semA).start()

            pltpu.make_async_copy(
                desc_hbm.at[pl.ds(r0, RSUB), :], buf_b, semB).wait()
            desc_compute(buf_b, sub + 1)

            @pl.when(p < SUB // 2 - 1)
            def _():
                pltpu.make_async_copy(
                    desc_hbm.at[pl.ds(r0 + (sub + 3) * RSUB, RSUB), :],
                    buf_b, semB).start()

            return carry

        lax.fori_loop(0, SUB // 2, pair, 0)

        pltpu.sync_copy(acc_v, out_hbm.at[pl.ds(r0, RPW)])

    return k(s2, title_flat, desc2d)


def _final_body(cat_ref, wcat_ref, bcat_ref, wh_ref, bout_ref, emb_ref,
                out_ref):
    h = lax.dot_general(
        cat_ref[...], wcat_ref[...],
        dimension_numbers=(((1,), (1,)), ((), ())),
        preferred_element_type=jnp.float32)
    h = jnp.maximum(h + bcat_ref[...], 0.0)
    o = lax.dot_general(
        wh_ref[...], h,
        dimension_numbers=(((1,), (1,)), ((), ())),
        preferred_element_type=jnp.float32)
    out_ref[...] = o + emb_ref[...] + bout_ref[0, 0]


def _final(Cat, W_cat, b_cat2, wh2, b_out2, emb2):
    BB = 2048
    return pl.pallas_call(
        _final_body,
        grid=(B // BB,),
        in_specs=[
            pl.BlockSpec((BB, C), lambda i: (i, 0)),
            pl.BlockSpec((H, C), lambda i: (0, 0)),
            pl.BlockSpec((1, H), lambda i: (0, 0)),
            pl.BlockSpec((1, H), lambda i: (0, 0)),
            pl.BlockSpec((1, 1), lambda i: (0, 0)),
            pl.BlockSpec((1, BB), lambda i: (0, i)),
        ],
        out_specs=pl.BlockSpec((1, BB), lambda i: (0, i)),
        out_shape=jax.ShapeDtypeStruct((1, B), jnp.float32),
    )(Cat, W_cat, b_cat2, wh2, b_out2, emb2)


def kernel(Title, FullDescription, Categorical, embedding_matrix, W_cat,
           b_cat, W_out, b_out):
    w2 = jnp.stack([W_out[0, :D] * (1.0 / LT),
                    W_out[0, D:2 * D] * (1.0 / LD)], axis=0)
    s2 = _compute_scores(embedding_matrix, w2)

    title_flat = Title.reshape(-1).astype(jnp.int32)
    emb_part = _sc_pool(s2, title_flat, FullDescription.astype(jnp.int32))

    out2 = _final(Categorical, W_cat, b_cat.reshape(1, H),
                  W_out[0:1, 2 * D:], b_out.reshape(1, 1),
                  emb_part.reshape(1, B))
    return out2.reshape(B)


# speedup vs baseline: 119.8653x; 1.4897x over previous
import functools

import jax
import jax.numpy as jnp
from jax import lax
from jax.experimental import pallas as pl
from jax.experimental.pallas import tpu as pltpu
from jax.experimental.pallas import tpu_sc as plsc

B = 16384
LT = 20
LD = 200
V = 100000
D = 64
C = 100
H = 128

NC = 2
NS = 16
NW = NC * NS
RPW = B // NW
SUB = 32
RSUB = RPW // SUB


def _scores_body(et_ref, w2_ref, out_ref):
    out_ref[...] = lax.dot_general(
        w2_ref[...], et_ref[...],
        dimension_numbers=(((1,), (0,)), ((), ())),
        preferred_element_type=jnp.float32)


def _compute_scores(ET, w2):
    VB = 8192
    return pl.pallas_call(
        _scores_body,
        grid=(pl.cdiv(V, VB),),
        in_specs=[
            pl.BlockSpec((D, VB), lambda i: (0, i)),
            pl.BlockSpec((2, D), lambda i: (0, 0)),
        ],
        out_specs=pl.BlockSpec((2, VB), lambda i: (0, i)),
        out_shape=jax.ShapeDtypeStruct((2, V), jnp.float32),
    )(ET, w2)


def _sc_pool(s2, titleT, desc_flat):
    mesh = plsc.VectorSubcoreMesh(core_axis_name="c", subcore_axis_name="s")

    @functools.partial(
        pl.kernel,
        mesh=mesh,
        out_type=jax.ShapeDtypeStruct((B,), jnp.float32),
        compiler_params=pltpu.CompilerParams(needs_layout_passes=False),
        scratch_types=[
            pltpu.VMEM((V,), jnp.float32),
            pltpu.VMEM((LT, RPW), jnp.int32),
            pltpu.VMEM((RSUB * LD,), jnp.int32),
            pltpu.VMEM((RSUB * LD,), jnp.int32),
            pltpu.VMEM((RPW,), jnp.float32),
            pltpu.SemaphoreType.DMA,
            pltpu.SemaphoreType.DMA,
            pltpu.SemaphoreType.DMA,
        ],
    )
    def k(s2_hbm, titleT_hbm, desc_hbm, out_hbm, table_v, tidx_v, buf_a,
          buf_b, acc_v, semT, semA, semB):
        w = lax.axis_index("c") * NS + lax.axis_index("s")
        r0 = w * RPW
        lanes = lax.iota(jnp.int32, 16)

        tcp = pltpu.make_async_copy(s2_hbm.at[0], table_v, semT)
        tcp.start()
        ti = pltpu.make_async_copy(
            titleT_hbm.at[:, pl.ds(r0, RPW)], tidx_v, semB)
        ti.start()
        pltpu.make_async_copy(
            desc_hbm.at[pl.ds(r0 * LD, RSUB * LD)], buf_a, semA).start()
        tcp.wait()
        ti.wait()

        def title_group(g, carry):
            acc0 = jnp.zeros((16,), jnp.float32)
            acc1 = jnp.zeros((16,), jnp.float32)
            for t in range(LT):
                iv = tidx_v[t, pl.ds(g * 16, 16)]
                vals = plsc.load_gather(table_v, [iv])
                if t % 2 == 0:
                    acc0 = acc0 + vals
                else:
                    acc1 = acc1 + vals
            acc_v[pl.ds(g * 16, 16)] = acc0 + acc1
            return carry

        lax.fori_loop(0, RPW // 16, title_group, 0)

        tB = pltpu.make_async_copy(s2_hbm.at[1], table_v, semT)
        tB.start()
        pltpu.make_async_copy(
            desc_hbm.at[pl.ds((r0 + RSUB) * LD, RSUB * LD)], buf_b,
            semB).start()
        tB.wait()

        def desc_chunk(buf, sub):
            base = lanes * LD
            accs = [jnp.zeros((16,), jnp.float32) for _ in range(4)]
            for t in range(LD):
                iv = plsc.load_gather(buf, [base + t])
                vals = plsc.load_gather(table_v, [iv])
                accs[t % 4] = accs[t % 4] + vals
            tot = (accs[0] + accs[1]) + (accs[2] + accs[3])
            off = sub * RSUB
            acc_v[pl.ds(off, 16)] = acc_v[pl.ds(off, 16)] + tot

        def pair(p, carry):
            sub = 2 * p
            pltpu.make_async_copy(
                desc_hbm.at[pl.ds(r0 * LD, RSUB * LD)], buf_a, semA).wait()
            desc_chunk(buf_a, sub)

            @pl.when(p < SUB // 2 - 1)
            def _():
                pltpu.make_async_copy(
                    desc_hbm.at[pl.ds((r0 + (sub + 2) * RSUB) * LD,
                                      RSUB * LD)], buf_a, semA).start()

            pltpu.make_async_copy(
                desc_hbm.at[pl.ds(r0 * LD, RSUB * LD)], buf_b, semB).wait()
            desc_chunk(buf_b, sub + 1)

            @pl.when(p < SUB // 2 - 1)
            def _():
                pltpu.make_async_copy(
                    desc_hbm.at[pl.ds((r0 + (sub + 3) * RSUB) * LD,
                                      RSUB * LD)], buf_b, semB).start()

            return carry

        lax.fori_loop(0, SUB // 2, pair, 0)

        pltpu.sync_copy(acc_v, out_hbm.at[pl.ds(r0, RPW)])

    return k(s2, titleT, desc_flat)


def _final_body(catT_ref, wcatT_ref, bcat_ref, wh_ref, bout_ref, emb_ref,
                out_ref):
    h = lax.dot_general(
        wcatT_ref[...], catT_ref[...],
        dimension_numbers=(((0,), (0,)), ((), ())),
        preferred_element_type=jnp.float32)
    h = jnp.maximum(h + bcat_ref[...], 0.0)
    o = lax.dot_general(
        wh_ref[...], h,
        dimension_numbers=(((1,), (0,)), ((), ())),
        preferred_element_type=jnp.float32)
    out_ref[...] = o + emb_ref[...] + bout_ref[0, 0]


def _final(CatT, WcatT, b_cat2, wh2, b_out2, emb2):
    BB = 2048
    return pl.pallas_call(
        _final_body,
        grid=(B // BB,),
        in_specs=[
            pl.BlockSpec((C, BB), lambda i: (0, i)),
            pl.BlockSpec((C, H), lambda i: (0, 0)),
            pl.BlockSpec((H, 1), lambda i: (0, 0)),
            pl.BlockSpec((1, H), lambda i: (0, 0)),
            pl.BlockSpec((1, 1), lambda i: (0, 0)),
            pl.BlockSpec((1, BB), lambda i: (0, i)),
        ],
        out_specs=pl.BlockSpec((1, BB), lambda i: (0, i)),
        out_shape=jax.ShapeDtypeStruct((1, B), jnp.float32),
    )(CatT, WcatT, b_cat2, wh2, b_out2, emb2)


def kernel(Title, FullDescription, Categorical, embedding_matrix, W_cat,
           b_cat, W_out, b_out):
    w2 = jnp.stack([W_out[0, :D] * (1.0 / LT),
                    W_out[0, D:2 * D] * (1.0 / LD)], axis=0)
    s2 = _compute_scores(embedding_matrix.T, w2)

    emb_part = _sc_pool(s2, Title.T.astype(jnp.int32),
                        FullDescription.reshape(-1).astype(jnp.int32))

    out2 = _final(Categorical.T, W_cat.T, b_cat.reshape(H, 1),
                  W_out[0:1, 2 * D:], b_out.reshape(1, 1),
                  emb_part.reshape(1, B))
    return out2.reshape(B)


# speedup vs baseline: 140.1068x; 1.1689x over previous
import functools

import jax
import jax.numpy as jnp
from jax import lax
from jax.experimental import pallas as pl
from jax.experimental.pallas import tpu as pltpu
from jax.experimental.pallas import tpu_sc as plsc

B = 16384
LT = 20
LD = 200
V = 100000
D = 64
C = 100
H = 128

NC = 2
NS = 16
NW = NC * NS
RPW = B // NW
SUB = 32
RSUB = RPW // SUB


def _scores_body(et_ref, w2_ref, out_ref):
    s = lax.dot_general(
        w2_ref[...], et_ref[...],
        dimension_numbers=(((1,), (0,)), ((), ())),
        preferred_element_type=jnp.float32)
    st = lax.bitcast_convert_type(
        s[0:1, :].astype(jnp.bfloat16), jnp.uint16).astype(jnp.uint32)
    sd = lax.bitcast_convert_type(
        s[1:2, :].astype(jnp.bfloat16), jnp.uint16).astype(jnp.uint32)
    packed = (st << 16) | sd
    out_ref[...] = lax.bitcast_convert_type(packed, jnp.int32)


def _compute_scores(ET, w2):
    VB = 8192
    return pl.pallas_call(
        _scores_body,
        grid=(pl.cdiv(V, VB),),
        in_specs=[
            pl.BlockSpec((D, VB), lambda i: (0, i)),
            pl.BlockSpec((2, D), lambda i: (0, 0)),
        ],
        out_specs=pl.BlockSpec((1, VB), lambda i: (0, i)),
        out_shape=jax.ShapeDtypeStruct((1, V), jnp.int32),
    )(ET, w2)


def _sc_pool(s2, titleT, desc_flat):
    mesh = plsc.VectorSubcoreMesh(core_axis_name="c", subcore_axis_name="s")

    @functools.partial(
        pl.kernel,
        mesh=mesh,
        out_type=jax.ShapeDtypeStruct((B,), jnp.float32),
        compiler_params=pltpu.CompilerParams(needs_layout_passes=False),
        scratch_types=[
            pltpu.VMEM((V,), jnp.int32),
            pltpu.VMEM((LT, RPW), jnp.int32),
            pltpu.VMEM((RSUB * LD,), jnp.int32),
            pltpu.VMEM((RSUB * LD,), jnp.int32),
            pltpu.VMEM((RPW,), jnp.float32),
            pltpu.SemaphoreType.DMA,
            pltpu.SemaphoreType.DMA,
            pltpu.SemaphoreType.DMA,
        ],
    )
    def k(s2_hbm, titleT_hbm, desc_hbm, out_hbm, table_v, tidx_v, buf_a,
          buf_b, acc_v, semT, semA, semB):
        w = lax.axis_index("c") * NS + lax.axis_index("s")
        r0 = w * RPW
        lanes = lax.iota(jnp.int32, 16)
        himask = jnp.full((16,), -65536, jnp.int32)

        tcp = pltpu.make_async_copy(s2_hbm.at[0], table_v, semT)
        tcp.start()
        ti = pltpu.make_async_copy(
            titleT_hbm.at[:, pl.ds(r0, RPW)], tidx_v, semT)
        ti.start()
        pltpu.make_async_copy(
            desc_hbm.at[pl.ds(r0 * LD, RSUB * LD)], buf_a, semA).start()
        pltpu.make_async_copy(
            desc_hbm.at[pl.ds((r0 + RSUB) * LD, RSUB * LD)], buf_b,
            semB).start()
        tcp.wait()
        ti.wait()

        def title_group(g, carry):
            acc0 = jnp.zeros((16,), jnp.float32)
            acc1 = jnp.zeros((16,), jnp.float32)
            for t in range(LT):
                iv = tidx_v[t, pl.ds(g * 16, 16)]
                word = plsc.load_gather(table_v, [iv])
                vals = plsc.bitcast(word & himask, jnp.float32)
                if t % 2 == 0:
                    acc0 = acc0 + vals
                else:
                    acc1 = acc1 + vals
            acc_v[pl.ds(g * 16, 16)] = acc0 + acc1
            return carry

        lax.fori_loop(0, RPW // 16, title_group, 0)

        def desc_chunk(buf, sub):
            base = lanes * LD
            accs = [jnp.zeros((16,), jnp.float32) for _ in range(4)]
            for t in range(LD):
                iv = plsc.load_gather(buf, [base + t])
                word = plsc.load_gather(table_v, [iv])
                vals = plsc.bitcast(word << 16, jnp.float32)
                accs[t % 4] = accs[t % 4] + vals
            tot = (accs[0] + accs[1]) + (accs[2] + accs[3])
            off = sub * RSUB
            acc_v[pl.ds(off, 16)] = acc_v[pl.ds(off, 16)] + tot

        def pair(p, carry):
            sub = 2 * p
            pltpu.make_async_copy(
                desc_hbm.at[pl.ds(r0 * LD, RSUB * LD)], buf_a, semA).wait()
            desc_chunk(buf_a, sub)

            @pl.when(p < SUB // 2 - 1)
            def _():
                pltpu.make_async_copy(
                    desc_hbm.at[pl.ds((r0 + (sub + 2) * RSUB) * LD,
                                      RSUB * LD)], buf_a, semA).start()

            pltpu.make_async_copy(
                desc_hbm.at[pl.ds(r0 * LD, RSUB * LD)], buf_b, semB).wait()
            desc_chunk(buf_b, sub + 1)

            @pl.when(p < SUB // 2 - 1)
            def _():
                pltpu.make_async_copy(
                    desc_hbm.at[pl.ds((r0 + (sub + 3) * RSUB) * LD,
                                      RSUB * LD)], buf_b, semB).start()

            return carry

        lax.fori_loop(0, SUB // 2, pair, 0)

        pltpu.sync_copy(acc_v, out_hbm.at[pl.ds(r0, RPW)])

    return k(s2, titleT, desc_flat)


def _final_body(catT_ref, wcatT_ref, bcat_ref, wh_ref, bout_ref, emb_ref,
                out_ref):
    h = lax.dot_general(
        wcatT_ref[...], catT_ref[...],
        dimension_numbers=(((0,), (0,)), ((), ())),
        preferred_element_type=jnp.float32)
    h = jnp.maximum(h + bcat_ref[...], 0.0)
    o = lax.dot_general(
        wh_ref[...], h,
        dimension_numbers=(((1,), (0,)), ((), ())),
        preferred_element_type=jnp.float32)
    out_ref[...] = o + emb_ref[...] + bout_ref[0, 0]


def _final(CatT, WcatT, b_cat2, wh2, b_out2, emb2):
    BB = 2048
    return pl.pallas_call(
        _final_body,
        grid=(B // BB,),
        in_specs=[
            pl.BlockSpec((C, BB), lambda i: (0, i)),
            pl.BlockSpec((C, H), lambda i: (0, 0)),
            pl.BlockSpec((H, 1), lambda i: (0, 0)),
            pl.BlockSpec((1, H), lambda i: (0, 0)),
            pl.BlockSpec((1, 1), lambda i: (0, 0)),
            pl.BlockSpec((1, BB), lambda i: (0, i)),
        ],
        out_specs=pl.BlockSpec((1, BB), lambda i: (0, i)),
        out_shape=jax.ShapeDtypeStruct((1, B), jnp.float32),
    )(CatT, WcatT, b_cat2, wh2, b_out2, emb2)


def kernel(Title, FullDescription, Categorical, embedding_matrix, W_cat,
           b_cat, W_out, b_out):
    w2 = jnp.stack([W_out[0, :D] * (1.0 / LT),
                    W_out[0, D:2 * D] * (1.0 / LD)], axis=0)
    s2 = _compute_scores(embedding_matrix.T, w2)

    emb_part = _sc_pool(s2, Title.T.astype(jnp.int32),
                        FullDescription.reshape(-1).astype(jnp.int32))

    out2 = _final(Categorical.T, W_cat.T, b_cat.reshape(H, 1),
                  W_out[0:1, 2 * D:], b_out.reshape(1, 1),
                  emb_part.reshape(1, B))
    return out2.reshape(B)


# speedup vs baseline: 140.2885x; 1.0013x over previous
import functools

import jax
import jax.numpy as jnp
from jax import lax
from jax.experimental import pallas as pl
from jax.experimental.pallas import tpu as pltpu
from jax.experimental.pallas import tpu_sc as plsc

B = 16384
LT = 20
LD = 200
V = 100000
D = 64
C = 100
H = 128

NC = 2
NS = 16
NW = NC * NS
RPW = B // NW
SUB = 32
RSUB = RPW // SUB


def _scores_body(et_ref, w2_ref, out_ref):
    s = lax.dot_general(
        w2_ref[...], et_ref[...],
        dimension_numbers=(((1,), (0,)), ((), ())),
        preferred_element_type=jnp.float32)
    st = lax.bitcast_convert_type(
        s[0:1, :].astype(jnp.bfloat16), jnp.uint16).astype(jnp.uint32)
    sd = lax.bitcast_convert_type(
        s[1:2, :].astype(jnp.bfloat16), jnp.uint16).astype(jnp.uint32)
    packed = (st << 16) | sd
    out_ref[...] = lax.bitcast_convert_type(packed, jnp.int32)


def _compute_scores(ET, w2):
    VB = 8192
    return pl.pallas_call(
        _scores_body,
        grid=(pl.cdiv(V, VB),),
        in_specs=[
            pl.BlockSpec((D, VB), lambda i: (0, i)),
            pl.BlockSpec((2, D), lambda i: (0, 0)),
        ],
        out_specs=pl.BlockSpec((1, VB), lambda i: (0, i)),
        out_shape=jax.ShapeDtypeStruct((1, V), jnp.int32),
    )(ET, w2)


def _sc_pool(s2, titleT, desc_flat):
    mesh = plsc.VectorSubcoreMesh(core_axis_name="c", subcore_axis_name="s")

    @functools.partial(
        pl.kernel,
        mesh=mesh,
        out_type=jax.ShapeDtypeStruct((B,), jnp.float32),
        compiler_params=pltpu.CompilerParams(needs_layout_passes=False),
        scratch_types=[
            pltpu.VMEM((V,), jnp.int32),
            pltpu.VMEM((LT, RPW), jnp.int32),
            pltpu.VMEM((RSUB * LD,), jnp.int32),
            pltpu.VMEM((RSUB * LD,), jnp.int32),
            pltpu.VMEM((RPW,), jnp.float32),
            pltpu.SemaphoreType.DMA,
            pltpu.SemaphoreType.DMA,
            pltpu.SemaphoreType.DMA,
        ],
    )
    def k(s2_hbm, titleT_hbm, desc_hbm, out_hbm, table_v, tidx_v, buf_a,
          buf_b, acc_v, semT, semA, semB):
        w = lax.axis_index("c") * NS + lax.axis_index("s")
        r0 = w * RPW
        lanes = lax.iota(jnp.int32, 16)
        himask = jnp.full((16,), -65536, jnp.int32)

        tcp = pltpu.make_async_copy(s2_hbm.at[0], table_v, semT)
        tcp.start()
        ti = pltpu.make_async_copy(
            titleT_hbm.at[:, pl.ds(r0, RPW)], tidx_v, semT)
        ti.start()
        pltpu.make_async_copy(
            desc_hbm.at[pl.ds(r0 * LD, RSUB * LD)], buf_a, semA).start()
        pltpu.make_async_copy(
            desc_hbm.at[pl.ds((r0 + RSUB) * LD, RSUB * LD)], buf_b,
            semB).start()
        tcp.wait()
        ti.wait()

        def title_group(g, carry):
            acc0 = jnp.zeros((16,), jnp.float32)
            acc1 = jnp.zeros((16,), jnp.float32)
            for t in range(LT):
                iv = tidx_v[t, pl.ds(g * 16, 16)]
                word = plsc.load_gather(table_v, [iv])
                vals = plsc.bitcast(word & himask, jnp.float32)
                if t % 2 == 0:
                    acc0 = acc0 + vals
                else:
                    acc1 = acc1 + vals
            acc_v[pl.ds(g * 16, 16)] = acc0 + acc1
            return carry

        lax.fori_loop(0, RPW // 16, title_group, 0)

        def desc_chunk(buf, sub):
            base = lanes * LD
            accs = [jnp.zeros((16,), jnp.float32) for _ in range(8)]
            for t in range(LD):
                iv = plsc.load_gather(buf, [base + t])
                word = plsc.load_gather(table_v, [iv])
                vals = plsc.bitcast(word << 16, jnp.float32)
                accs[t % 8] = accs[t % 8] + vals
            tot = (((accs[0] + accs[1]) + (accs[2] + accs[3]))
                   + ((accs[4] + accs[5]) + (accs[6] + accs[7])))
            off = sub * RSUB
            acc_v[pl.ds(off, 16)] = acc_v[pl.ds(off, 16)] + tot

        def pair(p, carry):
            sub = 2 * p
            pltpu.make_async_copy(
                desc_hbm.at[pl.ds(r0 * LD, RSUB * LD)], buf_a, semA).wait()
            desc_chunk(buf_a, sub)

            @pl.when(p < SUB // 2 - 1)
            def _():
                pltpu.make_async_copy(
                    desc_hbm.at[pl.ds((r0 + (sub + 2) * RSUB) * LD,
                                      RSUB * LD)], buf_a, semA).start()

            pltpu.make_async_copy(
                desc_hbm.at[pl.ds(r0 * LD, RSUB * LD)], buf_b, semB).wait()
            desc_chunk(buf_b, sub + 1)

            @pl.when(p < SUB // 2 - 1)
            def _():
                pltpu.make_async_copy(
                    desc_hbm.at[pl.ds((r0 + (sub + 3) * RSUB) * LD,
                                      RSUB * LD)], buf_b, semB).start()

            return carry

        lax.fori_loop(0, SUB // 2, pair, 0)

        pltpu.sync_copy(acc_v, out_hbm.at[pl.ds(r0, RPW)])

    return k(s2, titleT, desc_flat)


def _final_body(catT_ref, wcatT_ref, bcat_ref, wh_ref, bout_ref, emb_ref,
                out_ref):
    h = lax.dot_general(
        wcatT_ref[...], catT_ref[...],
        dimension_numbers=(((0,), (0,)), ((), ())),
        preferred_element_type=jnp.float32)
    h = jnp.maximum(h + bcat_ref[...], 0.0)
    o = lax.dot_general(
        wh_ref[...], h,
        dimension_numbers=(((1,), (0,)), ((), ())),
        preferred_element_type=jnp.float32)
    out_ref[...] = o + emb_ref[...] + bout_ref[0, 0]


def _final(CatT, WcatT, b_cat2, wh2, b_out2, emb2):
    BB = 2048
    return pl.pallas_call(
        _final_body,
        grid=(B // BB,),
        in_specs=[
            pl.BlockSpec((C, BB), lambda i: (0, i)),
            pl.BlockSpec((C, H), lambda i: (0, 0)),
            pl.BlockSpec((H, 1), lambda i: (0, 0)),
            pl.BlockSpec((1, H), lambda i: (0, 0)),
            pl.BlockSpec((1, 1), lambda i: (0, 0)),
            pl.BlockSpec((1, BB), lambda i: (0, i)),
        ],
        out_specs=pl.BlockSpec((1, BB), lambda i: (0, i)),
        out_shape=jax.ShapeDtypeStruct((1, B), jnp.float32),
    )(CatT, WcatT, b_cat2, wh2, b_out2, emb2)


def kernel(Title, FullDescription, Categorical, embedding_matrix, W_cat,
           b_cat, W_out, b_out):
    w2 = jnp.stack([W_out[0, :D] * (1.0 / LT),
                    W_out[0, D:2 * D] * (1.0 / LD)], axis=0)
    s2 = _compute_scores(embedding_matrix.T, w2)

    emb_part = _sc_pool(s2, Title.T.astype(jnp.int32),
                        FullDescription.reshape(-1).astype(jnp.int32))

    out2 = _final(Categorical.T, W_cat.T, b_cat.reshape(H, 1),
                  W_out[0:1, 2 * D:], b_out.reshape(1, 1),
                  emb_part.reshape(1, B))
    return out2.reshape(B)


# speedup vs baseline: 214.5350x; 1.5292x over previous
import functools

import jax
import jax.numpy as jnp
from jax import lax
from jax.experimental import pallas as pl
from jax.experimental.pallas import tpu as pltpu
from jax.experimental.pallas import tpu_sc as plsc

B = 16384
LT = 20
LD = 200
V = 100000
D = 64
C = 100
H = 128

NC = 2
NS = 16
NW = NC * NS
RPW = B // NW
DCH = 8
NCH = LD // DCH


def _scores_body(et_ref, w2_ref, out_ref):
    s = lax.dot_general(
        w2_ref[...], et_ref[...],
        dimension_numbers=(((1,), (0,)), ((), ())),
        preferred_element_type=jnp.float32)
    st = lax.bitcast_convert_type(
        s[0:1, :].astype(jnp.bfloat16), jnp.uint16).astype(jnp.uint32)
    sd = lax.bitcast_convert_type(
        s[1:2, :].astype(jnp.bfloat16), jnp.uint16).astype(jnp.uint32)
    packed = (st << 16) | sd
    out_ref[...] = lax.bitcast_convert_type(packed, jnp.int32)


def _compute_scores(ET, w2):
    VB = 8192
    return pl.pallas_call(
        _scores_body,
        grid=(pl.cdiv(V, VB),),
        in_specs=[
            pl.BlockSpec((D, VB), lambda i: (0, i)),
            pl.BlockSpec((2, D), lambda i: (0, 0)),
        ],
        out_specs=pl.BlockSpec((1, VB), lambda i: (0, i)),
        out_shape=jax.ShapeDtypeStruct((1, V), jnp.int32),
    )(ET, w2)


def _sc_pool(s2, titleT, descT):
    mesh = plsc.VectorSubcoreMesh(core_axis_name="c", subcore_axis_name="s")

    @functools.partial(
        pl.kernel,
        mesh=mesh,
        out_type=jax.ShapeDtypeStruct((B,), jnp.float32),
        compiler_params=pltpu.CompilerParams(needs_layout_passes=False),
        scratch_types=[
            pltpu.VMEM((V,), jnp.int32),
            pltpu.VMEM((LT, RPW), jnp.int32),
            pltpu.VMEM((DCH, RPW), jnp.int32),
            pltpu.VMEM((DCH, RPW), jnp.int32),
            pltpu.VMEM((RPW,), jnp.float32),
            pltpu.SemaphoreType.DMA,
            pltpu.SemaphoreType.DMA,
            pltpu.SemaphoreType.DMA,
        ],
    )
    def k(s2_hbm, titleT_hbm, descT_hbm, out_hbm, table_v, tidx_v, buf_a,
          buf_b, acc_v, semT, semA, semB):
        w = lax.axis_index("c") * NS + lax.axis_index("s")
        r0 = w * RPW
        himask = jnp.full((16,), -65536, jnp.int32)

        tcp = pltpu.make_async_copy(s2_hbm.at[0], table_v, semT)
        tcp.start()
        ti = pltpu.make_async_copy(
            titleT_hbm.at[:, pl.ds(r0, RPW)], tidx_v, semT)
        ti.start()
        pltpu.make_async_copy(
            descT_hbm.at[pl.ds(0, DCH), pl.ds(r0, RPW)], buf_a,
            semA).start()
        pltpu.make_async_copy(
            descT_hbm.at[pl.ds(DCH, DCH), pl.ds(r0, RPW)], buf_b,
            semB).start()
        tcp.wait()
        ti.wait()

        def title_group(g, carry):
            acc0 = jnp.zeros((16,), jnp.float32)
            acc1 = jnp.zeros((16,), jnp.float32)
            for t in range(LT):
                iv = tidx_v[t, pl.ds(g * 16, 16)]
                word = plsc.load_gather(table_v, [iv])
                vals = plsc.bitcast(word & himask, jnp.float32)
                if t % 2 == 0:
                    acc0 = acc0 + vals
                else:
                    acc1 = acc1 + vals
            acc_v[pl.ds(g * 16, 16)] = acc0 + acc1
            return carry

        lax.fori_loop(0, RPW // 16, title_group, 0)

        def desc_chunk(buf):
            def group(g, c2):
                acc0 = jnp.zeros((16,), jnp.float32)
                acc1 = jnp.zeros((16,), jnp.float32)
                for t in range(DCH):
                    iv = buf[t, pl.ds(g * 16, 16)]
                    word = plsc.load_gather(table_v, [iv])
                    vals = plsc.bitcast(word << 16, jnp.float32)
                    if t % 2 == 0:
                        acc0 = acc0 + vals
                    else:
                        acc1 = acc1 + vals
                acc_v[pl.ds(g * 16, 16)] = (acc_v[pl.ds(g * 16, 16)]
                                            + (acc0 + acc1))
                return c2

            lax.fori_loop(0, RPW // 16, group, 0)

        def pair(p, carry):
            c = 2 * p
            pltpu.make_async_copy(
                descT_hbm.at[pl.ds(0, DCH), pl.ds(r0, RPW)], buf_a,
                semA).wait()
            desc_chunk(buf_a)
            pltpu.make_async_copy(
                descT_hbm.at[pl.ds((c + 2) * DCH, DCH), pl.ds(r0, RPW)],
                buf_a, semA).start()

            pltpu.make_async_copy(
                descT_hbm.at[pl.ds(0, DCH), pl.ds(r0, RPW)], buf_b,
                semB).wait()
            desc_chunk(buf_b)

            @pl.when(p < NCH // 2 - 1)
            def _():
                pltpu.make_async_copy(
                    descT_hbm.at[pl.ds((c + 3) * DCH, DCH),
                                 pl.ds(r0, RPW)], buf_b, semB).start()

            return carry

        lax.fori_loop(0, NCH // 2, pair, 0)

        pltpu.make_async_copy(
            descT_hbm.at[pl.ds(0, DCH), pl.ds(r0, RPW)], buf_a,
            semA).wait()
        desc_chunk(buf_a)

        pltpu.sync_copy(acc_v, out_hbm.at[pl.ds(r0, RPW)])

    return k(s2, titleT, descT)


def _final_body(catT_ref, wcatT_ref, bcat_ref, wh_ref, bout_ref, emb_ref,
                out_ref):
    h = lax.dot_general(
        wcatT_ref[...], catT_ref[...],
        dimension_numbers=(((0,), (0,)), ((), ())),
        preferred_element_type=jnp.float32)
    h = jnp.maximum(h + bcat_ref[...], 0.0)
    o = lax.dot_general(
        wh_ref[...], h,
        dimension_numbers=(((1,), (0,)), ((), ())),
        preferred_element_type=jnp.float32)
    out_ref[...] = o + emb_ref[...] + bout_ref[0, 0]


def _final(CatT, WcatT, b_cat2, wh2, b_out2, emb2):
    BB = 2048
    return pl.pallas_call(
        _final_body,
        grid=(B // BB,),
        in_specs=[
            pl.BlockSpec((C, BB), lambda i: (0, i)),
            pl.BlockSpec((C, H), lambda i: (0, 0)),
            pl.BlockSpec((H, 1), lambda i: (0, 0)),
            pl.BlockSpec((1, H), lambda i: (0, 0)),
            pl.BlockSpec((1, 1), lambda i: (0, 0)),
            pl.BlockSpec((1, BB), lambda i: (0, i)),
        ],
        out_specs=pl.BlockSpec((1, BB), lambda i: (0, i)),
        out_shape=jax.ShapeDtypeStruct((1, B), jnp.float32),
    )(CatT, WcatT, b_cat2, wh2, b_out2, emb2)


def kernel(Title, FullDescription, Categorical, embedding_matrix, W_cat,
           b_cat, W_out, b_out):
    w2 = jnp.stack([W_out[0, :D] * (1.0 / LT),
                    W_out[0, D:2 * D] * (1.0 / LD)], axis=0)
    s2 = _compute_scores(embedding_matrix.T, w2)

    emb_part = _sc_pool(s2, Title.T.astype(jnp.int32),
                        FullDescription.T.astype(jnp.int32))

    out2 = _final(Categorical.T, W_cat.T, b_cat.reshape(H, 1),
                  W_out[0:1, 2 * D:], b_out.reshape(1, 1),
                  emb_part.reshape(1, B))
    return out2.reshape(B)


# speedup vs baseline: 228.6182x; 1.0656x over previous
import functools

import jax
import jax.numpy as jnp
from jax import lax
from jax.experimental import pallas as pl
from jax.experimental.pallas import tpu as pltpu
from jax.experimental.pallas import tpu_sc as plsc

B = 16384
LT = 20
LD = 200
V = 100000
D = 64
C = 100
H = 128

NC = 2
NS = 16
NW = NC * NS
RPW = B // NW
DCH = 8
NCH = LD // DCH


def _scores_body(et_ref, w2_ref, out_ref):
    s = lax.dot_general(
        w2_ref[...], et_ref[...],
        dimension_numbers=(((1,), (0,)), ((), ())),
        preferred_element_type=jnp.float32)
    st = lax.bitcast_convert_type(
        s[0:1, :].astype(jnp.bfloat16), jnp.uint16).astype(jnp.uint32)
    sd = lax.bitcast_convert_type(
        s[1:2, :].astype(jnp.bfloat16), jnp.uint16).astype(jnp.uint32)
    packed = (st << 16) | sd
    out_ref[...] = lax.bitcast_convert_type(packed, jnp.int32)


def _compute_scores(ET, w2):
    VB = 8192
    return pl.pallas_call(
        _scores_body,
        grid=(pl.cdiv(V, VB),),
        in_specs=[
            pl.BlockSpec((D, VB), lambda i: (0, i)),
            pl.BlockSpec((2, D), lambda i: (0, 0)),
        ],
        out_specs=pl.BlockSpec((1, VB), lambda i: (0, i)),
        out_shape=jax.ShapeDtypeStruct((1, V), jnp.int32),
    )(ET, w2)


def _sc_pool(s2, titleT, descT):
    mesh = plsc.VectorSubcoreMesh(core_axis_name="c", subcore_axis_name="s")

    @functools.partial(
        pl.kernel,
        mesh=mesh,
        out_type=jax.ShapeDtypeStruct((B,), jnp.float32),
        compiler_params=pltpu.CompilerParams(needs_layout_passes=False),
        scratch_types=[
            pltpu.VMEM((V,), jnp.int32),
            pltpu.VMEM((LT, RPW), jnp.int32),
            pltpu.VMEM((DCH, RPW), jnp.int32),
            pltpu.VMEM((DCH, RPW), jnp.int32),
            pltpu.VMEM((RPW,), jnp.float32),
            pltpu.SemaphoreType.DMA,
            pltpu.SemaphoreType.DMA,
            pltpu.SemaphoreType.DMA,
        ],
    )
    def k(s2_hbm, titleT_hbm, descT_hbm, out_hbm, table_v, tidx_v, buf_a,
          buf_b, acc_v, semT, semA, semB):
        w = lax.axis_index("c") * NS + lax.axis_index("s")
        r0 = w * RPW
        himask = jnp.full((16,), -65536, jnp.int32)

        tcp = pltpu.make_async_copy(s2_hbm.at[0], table_v, semT)
        tcp.start()
        ti = pltpu.make_async_copy(
            titleT_hbm.at[:, pl.ds(r0, RPW)], tidx_v, semT)
        ti.start()
        pltpu.make_async_copy(
            descT_hbm.at[pl.ds(0, DCH), pl.ds(r0, RPW)], buf_a,
            semA).start()
        pltpu.make_async_copy(
            descT_hbm.at[pl.ds(DCH, DCH), pl.ds(r0, RPW)], buf_b,
            semB).start()
        tcp.wait()
        ti.wait()

        def title_group(g, carry):
            acc0 = jnp.zeros((16,), jnp.float32)
            acc1 = jnp.zeros((16,), jnp.float32)
            for t in range(LT):
                iv = tidx_v[t, pl.ds(g * 16, 16)]
                word = plsc.load_gather(table_v, [iv])
                vals = plsc.bitcast(word & himask, jnp.float32)
                if t % 2 == 0:
                    acc0 = acc0 + vals
                else:
                    acc1 = acc1 + vals
            acc_v[pl.ds(g * 16, 16)] = acc0 + acc1
            return carry

        lax.fori_loop(0, RPW // 16, title_group, 0)

        def desc_chunk(buf):
            def group(g, c2):
                acc0 = jnp.zeros((16,), jnp.float32)
                acc1 = jnp.zeros((16,), jnp.float32)
                for t in range(DCH):
                    iv = buf[t, pl.ds(g * 16, 16)]
                    word = plsc.load_gather(table_v, [iv])
                    vals = plsc.bitcast(word << 16, jnp.float32)
                    if t % 2 == 0:
                        acc0 = acc0 + vals
                    else:
                        acc1 = acc1 + vals
                acc_v[pl.ds(g * 16, 16)] = (acc_v[pl.ds(g * 16, 16)]
                                            + (acc0 + acc1))
                return c2

            lax.fori_loop(0, RPW // 16, group, 0)

        def pair(p, carry):
            c = 2 * p
            pltpu.make_async_copy(
                descT_hbm.at[pl.ds(0, DCH), pl.ds(r0, RPW)], buf_a,
                semA).wait()
            desc_chunk(buf_a)
            pltpu.make_async_copy(
                descT_hbm.at[pl.ds((c + 2) * DCH, DCH), pl.ds(r0, RPW)],
                buf_a, semA).start()

            pltpu.make_async_copy(
                descT_hbm.at[pl.ds(0, DCH), pl.ds(r0, RPW)], buf_b,
                semB).wait()
            desc_chunk(buf_b)

            @pl.when(p < NCH // 2 - 1)
            def _():
                pltpu.make_async_copy(
                    descT_hbm.at[pl.ds((c + 3) * DCH, DCH),
                                 pl.ds(r0, RPW)], buf_b, semB).start()

            return carry

        lax.fori_loop(0, NCH // 2, pair, 0)

        pltpu.make_async_copy(
            descT_hbm.at[pl.ds(0, DCH), pl.ds(r0, RPW)], buf_a,
            semA).wait()
        desc_chunk(buf_a)

        pltpu.sync_copy(acc_v, out_hbm.at[pl.ds(r0, RPW)])

    return k(s2, titleT, descT)


def _cat_body(catT_ref, wcatT_ref, bcat_ref, wh_ref, bout_ref, out_ref):
    h = lax.dot_general(
        wcatT_ref[...], catT_ref[...],
        dimension_numbers=(((0,), (0,)), ((), ())),
        preferred_element_type=jnp.float32)
    h = jnp.maximum(h + bcat_ref[...], 0.0)
    o = lax.dot_general(
        wh_ref[...], h,
        dimension_numbers=(((1,), (0,)), ((), ())),
        preferred_element_type=jnp.float32)
    out_ref[...] = o + bout_ref[0, 0]


def _cat_part(CatT, WcatT, b_cat2, wh2, b_out2):
    BB = 2048
    return pl.pallas_call(
        _cat_body,
        grid=(B // BB,),
        in_specs=[
            pl.BlockSpec((C, BB), lambda i: (0, i)),
            pl.BlockSpec((C, H), lambda i: (0, 0)),
            pl.BlockSpec((H, 1), lambda i: (0, 0)),
            pl.BlockSpec((1, H), lambda i: (0, 0)),
            pl.BlockSpec((1, 1), lambda i: (0, 0)),
        ],
        out_specs=pl.BlockSpec((1, BB), lambda i: (0, i)),
        out_shape=jax.ShapeDtypeStruct((1, B), jnp.float32),
    )(CatT, WcatT, b_cat2, wh2, b_out2)


def _add_body(a_ref, b_ref, out_ref):
    out_ref[...] = a_ref[...] + b_ref[...]


def _add_final(cat2, emb2):
    BB = 8192
    return pl.pallas_call(
        _add_body,
        grid=(B // BB,),
        in_specs=[
            pl.BlockSpec((1, BB), lambda i: (0, i)),
            pl.BlockSpec((1, BB), lambda i: (0, i)),
        ],
        out_specs=pl.BlockSpec((1, BB), lambda i: (0, i)),
        out_shape=jax.ShapeDtypeStruct((1, B), jnp.float32),
    )(cat2, emb2)


def kernel(Title, FullDescription, Categorical, embedding_matrix, W_cat,
           b_cat, W_out, b_out):
    w2 = jnp.stack([W_out[0, :D] * (1.0 / LT),
                    W_out[0, D:2 * D] * (1.0 / LD)], axis=0)
    s2 = _compute_scores(embedding_matrix.T, w2)

    emb_part = _sc_pool(s2, Title.T.astype(jnp.int32),
                        FullDescription.T.astype(jnp.int32))

    cat2 = _cat_part(Categorical.T, W_cat.T, b_cat.reshape(H, 1),
                     W_out[0:1, 2 * D:], b_out.reshape(1, 1))
    out2 = _add_final(cat2, emb_part.reshape(1, B))
    return out2.reshape(B)


# speedup vs baseline: 238.8031x; 1.0446x over previous
import functools

import jax
import jax.numpy as jnp
from jax import lax
from jax.experimental import pallas as pl
from jax.experimental.pallas import tpu as pltpu
from jax.experimental.pallas import tpu_sc as plsc

B = 16384
LT = 20
LD = 200
V = 100000
D = 64
C = 100
H = 128

NC = 2
NS = 16
NW = NC * NS
RPW = B // NW
DCH = 8
NCH = LD // DCH


def _scores_body(et_ref, w2_ref, out_ref):
    s = lax.dot_general(
        w2_ref[...], et_ref[...],
        dimension_numbers=(((1,), (0,)), ((), ())),
        preferred_element_type=jnp.float32)
    st = lax.bitcast_convert_type(
        s[0:1, :].astype(jnp.bfloat16), jnp.uint16).astype(jnp.uint32)
    sd = lax.bitcast_convert_type(
        s[1:2, :].astype(jnp.bfloat16), jnp.uint16).astype(jnp.uint32)
    packed = (st << 16) | sd
    out_ref[...] = lax.bitcast_convert_type(packed, jnp.int32)


def _compute_scores(ET, w2):
    VB = 12800
    return pl.pallas_call(
        _scores_body,
        grid=(pl.cdiv(V, VB),),
        in_specs=[
            pl.BlockSpec((D, VB), lambda i: (0, i)),
            pl.BlockSpec((2, D), lambda i: (0, 0)),
        ],
        out_specs=pl.BlockSpec((1, VB), lambda i: (0, i)),
        out_shape=jax.ShapeDtypeStruct((1, V), jnp.int32),
    )(ET, w2)


def _sc_pool(s2, titleT, descT):
    mesh = plsc.VectorSubcoreMesh(core_axis_name="c", subcore_axis_name="s")

    @functools.partial(
        pl.kernel,
        mesh=mesh,
        out_type=jax.ShapeDtypeStruct((B,), jnp.float32),
        compiler_params=pltpu.CompilerParams(needs_layout_passes=False),
        scratch_types=[
            pltpu.VMEM((V,), jnp.int32),
            pltpu.VMEM((LT, RPW), jnp.int32),
            pltpu.VMEM((DCH, RPW), jnp.int32),
            pltpu.VMEM((DCH, RPW), jnp.int32),
            pltpu.VMEM((RPW,), jnp.float32),
            pltpu.SemaphoreType.DMA,
            pltpu.SemaphoreType.DMA,
            pltpu.SemaphoreType.DMA,
        ],
    )
    def k(s2_hbm, titleT_hbm, descT_hbm, out_hbm, table_v, tidx_v, buf_a,
          buf_b, acc_v, semT, semA, semB):
        w = lax.axis_index("c") * NS + lax.axis_index("s")
        r0 = w * RPW
        himask = jnp.full((16,), -65536, jnp.int32)

        tcp = pltpu.make_async_copy(s2_hbm.at[0], table_v, semT)
        tcp.start()
        ti = pltpu.make_async_copy(
            titleT_hbm.at[:, pl.ds(r0, RPW)], tidx_v, semT)
        ti.start()
        pltpu.make_async_copy(
            descT_hbm.at[pl.ds(0, DCH), pl.ds(r0, RPW)], buf_a,
            semA).start()
        pltpu.make_async_copy(
            descT_hbm.at[pl.ds(DCH, DCH), pl.ds(r0, RPW)], buf_b,
            semB).start()
        tcp.wait()
        ti.wait()

        def title_group(g, carry):
            acc0 = jnp.zeros((16,), jnp.float32)
            acc1 = jnp.zeros((16,), jnp.float32)
            for t in range(LT):
                iv = tidx_v[t, pl.ds(g * 16, 16)]
                word = plsc.load_gather(table_v, [iv])
                vals = plsc.bitcast(word & himask, jnp.float32)
                if t % 2 == 0:
                    acc0 = acc0 + vals
                else:
                    acc1 = acc1 + vals
            acc_v[pl.ds(g * 16, 16)] = acc0 + acc1
            return carry

        lax.fori_loop(0, RPW // 16, title_group, 0)

        def desc_chunk(buf):
            def group(g, c2):
                acc0 = jnp.zeros((16,), jnp.float32)
                acc1 = jnp.zeros((16,), jnp.float32)
                for t in range(DCH):
                    iv = buf[t, pl.ds(g * 16, 16)]
                    word = plsc.load_gather(table_v, [iv])
                    vals = plsc.bitcast(word << 16, jnp.float32)
                    if t % 2 == 0:
                        acc0 = acc0 + vals
                    else:
                        acc1 = acc1 + vals
                acc_v[pl.ds(g * 16, 16)] = (acc_v[pl.ds(g * 16, 16)]
                                            + (acc0 + acc1))
                return c2

            lax.fori_loop(0, RPW // 16, group, 0)

        def pair(p, carry):
            c = 2 * p
            pltpu.make_async_copy(
                descT_hbm.at[pl.ds(0, DCH), pl.ds(r0, RPW)], buf_a,
                semA).wait()
            desc_chunk(buf_a)
            pltpu.make_async_copy(
                descT_hbm.at[pl.ds((c + 2) * DCH, DCH), pl.ds(r0, RPW)],
                buf_a, semA).start()

            pltpu.make_async_copy(
                descT_hbm.at[pl.ds(0, DCH), pl.ds(r0, RPW)], buf_b,
                semB).wait()
            desc_chunk(buf_b)

            @pl.when(p < NCH // 2 - 1)
            def _():
                pltpu.make_async_copy(
                    descT_hbm.at[pl.ds((c + 3) * DCH, DCH),
                                 pl.ds(r0, RPW)], buf_b, semB).start()

            return carry

        lax.fori_loop(0, NCH // 2, pair, 0)

        pltpu.make_async_copy(
            descT_hbm.at[pl.ds(0, DCH), pl.ds(r0, RPW)], buf_a,
            semA).wait()
        desc_chunk(buf_a)

        pltpu.sync_copy(acc_v, out_hbm.at[pl.ds(r0, RPW)])

    return k(s2, titleT, descT)


def _cat_body(catT_ref, wcatT_ref, bcat_ref, wh_ref, bout_ref, out_ref):
    h = lax.dot_general(
        wcatT_ref[...], catT_ref[...],
        dimension_numbers=(((0,), (0,)), ((), ())),
        preferred_element_type=jnp.float32)
    h = jnp.maximum(h + bcat_ref[...], 0.0)
    o = lax.dot_general(
        wh_ref[...], h,
        dimension_numbers=(((1,), (0,)), ((), ())),
        preferred_element_type=jnp.float32)
    out_ref[...] = o + bout_ref[0, 0]


def _cat_part(CatT, WcatT, b_cat2, wh2, b_out2):
    BB = 2048
    return pl.pallas_call(
        _cat_body,
        grid=(B // BB,),
        in_specs=[
            pl.BlockSpec((C, BB), lambda i: (0, i)),
            pl.BlockSpec((C, H), lambda i: (0, 0)),
            pl.BlockSpec((H, 1), lambda i: (0, 0)),
            pl.BlockSpec((1, H), lambda i: (0, 0)),
            pl.BlockSpec((1, 1), lambda i: (0, 0)),
        ],
        out_specs=pl.BlockSpec((1, BB), lambda i: (0, i)),
        out_shape=jax.ShapeDtypeStruct((1, B), jnp.float32),
    )(CatT, WcatT, b_cat2, wh2, b_out2)


def _add_body(a_ref, b_ref, out_ref):
    out_ref[...] = a_ref[...] + b_ref[...]


def _add_final(cat2, emb2):
    BB = 8192
    return pl.pallas_call(
        _add_body,
        grid=(B // BB,),
        in_specs=[
            pl.BlockSpec((1, BB), lambda i: (0, i)),
            pl.BlockSpec((1, BB), lambda i: (0, i)),
        ],
        out_specs=pl.BlockSpec((1, BB), lambda i: (0, i)),
        out_shape=jax.ShapeDtypeStruct((1, B), jnp.float32),
    )(cat2, emb2)


def kernel(Title, FullDescription, Categorical, embedding_matrix, W_cat,
           b_cat, W_out, b_out):
    w2 = jnp.stack([W_out[0, :D] * (1.0 / LT),
                    W_out[0, D:2 * D] * (1.0 / LD)], axis=0)
    s2 = _compute_scores(embedding_matrix.T, w2)

    emb_part = _sc_pool(s2, Title.T.astype(jnp.int32),
                        FullDescription.T.astype(jnp.int32))

    cat2 = _cat_part(Categorical.T, W_cat.T, b_cat.reshape(H, 1),
                     W_out[0:1, 2 * D:], b_out.reshape(1, 1))
    out2 = _add_final(cat2, emb_part.reshape(1, B))
    return out2.reshape(B)
